# baseline (speedup 1.0000x reference)
import jax
import jax.numpy as jnp
from jax.experimental import pallas as pl
from jax.experimental.pallas import tpu as pltpu

_ALPHA = 1.0
_BETA = 0.5
_GAMMA = 0.5
_EPS = 1e-12
_BI = 1024


def _split(x):
    hi = x.astype(jnp.bfloat16)
    lo = (x - hi.astype(jnp.float32)).astype(jnp.bfloat16)
    return hi, lo


def _loss_kernel(scal_ref, pred_ref, y_ref, mpos_ref, mvel_ref, volsb_ref,
                 ystd_row_ref, ymean_row_ref,
                 out_ref, xta_s, xtpq_s, pos_s, vel_s):
    i = pl.program_id(0)
    n_total = y_ref.shape[0]

    rho_0 = scal_ref[0, 0]
    h = scal_ref[0, 1]
    dt = scal_ref[0, 2]
    nbp = scal_ref[0, 3].astype(jnp.int32)
    hinv = 1.0 / h
    dtinv = 1.0 / dt

    @pl.when(i == 0)
    def _():
        y_inv = y_ref[...] * ystd_row_ref[...] + ymean_row_ref[...]
        riota = jax.lax.broadcasted_iota(jnp.int32, (n_total, 1), 0)
        free = riota >= nbp
        z = jnp.zeros_like(y_inv)
        pos = mpos_ref[...] + jnp.where(free, y_inv, z)
        vel = mvel_ref[...] + jnp.where(free, y_inv * dtinv, z)
        pos_s[...] = pos
        vel_s[...] = vel
        pT = jnp.transpose(pos)
        vT = jnp.transpose(vel)
        rje = (pT[0:1, :] * pT[0:1, :] + pT[1:2, :] * pT[1:2, :]
               + pT[2:3, :] * pT[2:3, :]) + _EPS
        msj = -(pT[0:1, :] * vT[0:1, :] + pT[1:2, :] * vT[1:2, :]
                + pT[2:3, :] * vT[2:3, :])
        onesr = jnp.ones((1, n_total), jnp.bfloat16)
        rjh, rjl = _split(rje)
        sjh, sjl = _split(msj)
        xh, xl = _split(pT)
        vh, vl = _split(vT)
        xta_s[...] = jnp.concatenate(
            [onesr, onesr, rjh, rjl, xh, xl, xh], axis=0)
        xtpq_s[...] = jnp.concatenate(
            [vh, vl, vh, xh, xl, xh, onesr, onesr, sjh, sjl], axis=0)
        out_ref[...] = jnp.zeros((1, 1), jnp.float32)

    rows = pl.ds(i * _BI, _BI)
    xb = pos_s[rows, :]
    vb = vel_s[rows, :]

    ri = jnp.sum(xb * xb, axis=1, keepdims=True)
    msi = -jnp.sum(xb * vb, axis=1, keepdims=True)
    rih, ril = _split(ri)
    sih, sil = _split(msi)
    onesc = jnp.ones((_BI, 1), jnp.bfloat16)
    xbh, xbl = _split(xb)
    vbh, vbl = _split(vb)
    m2xbh = -2.0 * xbh
    m2xbl = -2.0 * xbl
    xia = jnp.concatenate(
        [rih, ril, onesc, onesc, m2xbh, m2xbh, m2xbl], axis=1)
    xipq = jnp.concatenate(
        [xbh, xbh, xbl, vbh, vbh, vbl, sih, sil, onesc, onesc], axis=1)

    D = jnp.dot(xia, xta_s[...], preferred_element_type=jnp.float32)
    dot = jnp.dot(xipq, xtpq_s[...], preferred_element_type=jnp.float32)

    d2p = jnp.maximum(D, _EPS)
    rinv = jax.lax.rsqrt(d2p)
    d = d2p * rinv
    q = (d * hinv).astype(jnp.bfloat16)

    q2 = q * q
    near = q <= 0.5
    u = jnp.maximum(1.0 - q, 0.0)
    u2 = u * u

    w_near = 6.0 * (q2 * (q - 1.0)) + 1.0
    w_far = (2.0 * u) * u2
    Wt = jnp.where(near, w_near, w_far)

    g_near = 18.0 * q2 - 12.0 * q
    g_far = -6.0 * u2
    Gt = jnp.where(near, g_near, g_far)

    Tt = (Gt * dot.astype(jnp.bfloat16)) * rinv.astype(jnp.bfloat16)

    volsb = volsb_ref[...]
    S2 = jnp.dot(Wt, volsb, preferred_element_type=jnp.float32)
    S3 = jnp.dot(Tt, volsb, preferred_element_type=jnp.float32)

    rho = rho_0 * S2
    cmp = rho / rho_0 - 1.0
    b2 = jnp.sum(jnp.abs(cmp))
    div = (rho_0 * hinv) * S3
    b3 = jnp.sum(jnp.abs(div))

    dy = y_ref[rows, :] - pred_ref[rows, :]
    b1 = jnp.sum(dy * dy)

    contrib = (_ALPHA * b1 + _BETA * b2 + _GAMMA * b3) / n_total
    out_ref[...] += jnp.reshape(contrib, (1, 1))


def kernel(pred, y, mid_pos, mid_vel, vol, rho_0, h, dt, y_mean, y_std,
           num_boundary_particles):
    n = pred.shape[0]
    f32 = jnp.float32
    sigma = 8.0 / (f32(jnp.pi) * h * h * h)
    volsb = (vol * sigma).astype(jnp.bfloat16).reshape(n, 1)
    scal = jnp.stack([jnp.asarray(rho_0, f32), jnp.asarray(h, f32),
                      jnp.asarray(dt, f32),
                      jnp.asarray(num_boundary_particles, f32)]).reshape(1, 4)

    full_n3 = pl.BlockSpec((n, 3), lambda i: (0, 0))

    out = pl.pallas_call(
        _loss_kernel,
        grid=(n // _BI,),
        in_specs=[
            pl.BlockSpec(memory_space=pltpu.SMEM),
            full_n3, full_n3, full_n3, full_n3,
            pl.BlockSpec((n, 1), lambda i: (0, 0)),
            pl.BlockSpec((1, 3), lambda i: (0, 0)),
            pl.BlockSpec((1, 3), lambda i: (0, 0)),
        ],
        out_specs=pl.BlockSpec((1, 1), lambda i: (0, 0)),
        out_shape=jax.ShapeDtypeStruct((1, 1), jnp.float32),
        scratch_shapes=[
            pltpu.VMEM((13, n), jnp.bfloat16),
            pltpu.VMEM((22, n), jnp.bfloat16),
            pltpu.VMEM((n, 3), jnp.float32),
            pltpu.VMEM((n, 3), jnp.float32),
        ],
    )(scal, pred, y, mid_pos, mid_vel, volsb,
      y_std.reshape(1, 3), y_mean.reshape(1, 3))
    return out.reshape(())


# speedup vs baseline: 1.0748x; 1.0748x over previous
import jax
import jax.numpy as jnp
from jax.experimental import pallas as pl
from jax.experimental.pallas import tpu as pltpu

_ALPHA = 1.0
_BETA = 0.5
_GAMMA = 0.5
_EPS = 1e-12
_BI = 1024


def _split(x):
    hi = x.astype(jnp.bfloat16)
    lo = (x - hi.astype(jnp.float32)).astype(jnp.bfloat16)
    return hi, lo


def _loss_kernel(scal_ref, pred_ref, y_ref, mpos_ref, mvel_ref, volsb_ref,
                 ystd_row_ref, ymean_row_ref,
                 out_ref, xta_s, xtpq_s, rje_s, sj_s, pos_s, vel_s):
    i = pl.program_id(0)
    n_total = y_ref.shape[0]

    rho_0 = scal_ref[0, 0]
    h = scal_ref[0, 1]
    dt = scal_ref[0, 2]
    nbp = scal_ref[0, 3].astype(jnp.int32)
    hinv = 1.0 / h
    dtinv = 1.0 / dt

    @pl.when(i == 0)
    def _():
        y_inv = y_ref[...] * ystd_row_ref[...] + ymean_row_ref[...]
        riota = jax.lax.broadcasted_iota(jnp.int32, (n_total, 1), 0)
        free = riota >= nbp
        z = jnp.zeros_like(y_inv)
        pos = mpos_ref[...] + jnp.where(free, y_inv, z)
        vel = mvel_ref[...] + jnp.where(free, y_inv * dtinv, z)
        pos_s[...] = pos
        vel_s[...] = vel
        pT = jnp.transpose(pos)
        vT = jnp.transpose(vel)
        rje_s[...] = (pT[0:1, :] * pT[0:1, :] + pT[1:2, :] * pT[1:2, :]
                      + pT[2:3, :] * pT[2:3, :]) + _EPS
        sj_s[...] = (pT[0:1, :] * vT[0:1, :] + pT[1:2, :] * vT[1:2, :]
                     + pT[2:3, :] * vT[2:3, :])
        xh, xl = _split(pT)
        vh, vl = _split(vT)
        xta_s[...] = jnp.concatenate([xh, xl, xh], axis=0)
        xtpq_s[...] = jnp.concatenate([vh, vl, vh, xh, xl, xh], axis=0)
        out_ref[...] = jnp.zeros((1, 1), jnp.float32)

    rows = pl.ds(i * _BI, _BI)
    xb = pos_s[rows, :]
    vb = vel_s[rows, :]

    ri = jnp.sum(xb * xb, axis=1, keepdims=True)
    si = jnp.sum(xb * vb, axis=1, keepdims=True)
    xbh, xbl = _split(xb)
    vbh, vbl = _split(vb)
    xia = jnp.concatenate([xbh, xbh, xbl], axis=1)
    xipq = jnp.concatenate([xbh, xbh, xbl, vbh, vbh, vbl], axis=1)

    A = jnp.dot(xia, xta_s[...], preferred_element_type=jnp.float32)
    PQ = jnp.dot(xipq, xtpq_s[...], preferred_element_type=jnp.float32)

    d2p = jnp.maximum((ri - (A + A)) + rje_s[...], _EPS)
    rinv = jax.lax.rsqrt(d2p)
    d = d2p * rinv
    q = (d * hinv).astype(jnp.bfloat16)

    q2 = q * q
    near = q <= 0.5
    u = jnp.maximum(1.0 - q, 0.0)
    u2 = u * u

    w_near = 6.0 * (q2 * (q - 1.0)) + 1.0
    w_far = (2.0 * u) * u2
    Wt = jnp.where(near, w_near, w_far)

    g_near = 18.0 * q2 - 12.0 * q
    g_far = -6.0 * u2
    Gt = jnp.where(near, g_near, g_far)

    dot = PQ - si - sj_s[...]
    Tt = (Gt * dot.astype(jnp.bfloat16)) * rinv.astype(jnp.bfloat16)

    volsb = volsb_ref[...]
    S2 = jnp.dot(Wt, volsb, preferred_element_type=jnp.float32)
    S3 = jnp.dot(Tt, volsb, preferred_element_type=jnp.float32)

    rho = rho_0 * S2
    cmp = rho / rho_0 - 1.0
    b2 = jnp.sum(jnp.abs(cmp))
    div = (rho_0 * hinv) * S3
    b3 = jnp.sum(jnp.abs(div))

    dy = y_ref[rows, :] - pred_ref[rows, :]
    b1 = jnp.sum(dy * dy)

    contrib = (_ALPHA * b1 + _BETA * b2 + _GAMMA * b3) / n_total
    out_ref[...] += jnp.reshape(contrib, (1, 1))


def kernel(pred, y, mid_pos, mid_vel, vol, rho_0, h, dt, y_mean, y_std,
           num_boundary_particles):
    n = pred.shape[0]
    f32 = jnp.float32
    sigma = 8.0 / (f32(jnp.pi) * h * h * h)
    volsb = (vol * sigma).astype(jnp.bfloat16).reshape(n, 1)
    scal = jnp.stack([jnp.asarray(rho_0, f32), jnp.asarray(h, f32),
                      jnp.asarray(dt, f32),
                      jnp.asarray(num_boundary_particles, f32)]).reshape(1, 4)

    full_n3 = pl.BlockSpec((n, 3), lambda i: (0, 0))

    out = pl.pallas_call(
        _loss_kernel,
        grid=(n // _BI,),
        in_specs=[
            pl.BlockSpec(memory_space=pltpu.SMEM),
            full_n3, full_n3, full_n3, full_n3,
            pl.BlockSpec((n, 1), lambda i: (0, 0)),
            pl.BlockSpec((1, 3), lambda i: (0, 0)),
            pl.BlockSpec((1, 3), lambda i: (0, 0)),
        ],
        out_specs=pl.BlockSpec((1, 1), lambda i: (0, 0)),
        out_shape=jax.ShapeDtypeStruct((1, 1), jnp.float32),
        scratch_shapes=[
            pltpu.VMEM((9, n), jnp.bfloat16),
            pltpu.VMEM((18, n), jnp.bfloat16),
            pltpu.VMEM((1, n), jnp.float32),
            pltpu.VMEM((1, n), jnp.float32),
            pltpu.VMEM((n, 3), jnp.float32),
            pltpu.VMEM((n, 3), jnp.float32),
        ],
    )(scal, pred, y, mid_pos, mid_vel, volsb,
      y_std.reshape(1, 3), y_mean.reshape(1, 3))
    return out.reshape(())
